# baseline (speedup 1.0000x reference)
import functools

import jax
import jax.numpy as jnp
from jax import lax
from jax.experimental import pallas as pl
from jax.experimental.pallas import tpu as pltpu
from jax.experimental.pallas import tpu_sc as plsc

N = 10000
E = 320000
D1 = 128
D2 = 40

NC = 2
NS = 16
NW = NC * NS
E_PER_W = E // NW
ROWS_PER_TILE = 624
REM_ROWS = N - NS * ROWS_PER_TILE

_CHUNK = {128: 40, 40: 80}
_NBUF = {128: 5, 40: 5}


@functools.lru_cache(maxsize=None)
def _make_sc_segsum(D):
    CHUNK = _CHUNK[D]
    NBUF = _NBUF[D]
    NCHUNK = E_PER_W // CHUNK
    mesh = plsc.VectorSubcoreMesh(core_axis_name="c", subcore_axis_name="s",
                                  num_cores=NC, num_subcores=NS)

    @functools.partial(
        pl.kernel,
        out_type=(
            jax.ShapeDtypeStruct((N, D), jnp.float32),
            jax.ShapeDtypeStruct((N, D), jnp.float32),
        ),
        mesh=mesh,
        compiler_params=pltpu.CompilerParams(use_tc_tiling_on_sc=False),
        scratch_types=[
            pltpu.VMEM((NCHUNK, CHUNK), jnp.int32),
            pltpu.VMEM((NCHUNK, CHUNK), jnp.int32),
            [pltpu.VMEM((CHUNK, D), jnp.float32) for _ in range(NBUF)],
            pltpu.VMEM_SHARED((N, D), jnp.float32),
            [pltpu.SemaphoreType.DMA for _ in range(NBUF)],
            [pltpu.SemaphoreType.DMA for _ in range(NBUF)],
            pltpu.SemaphoreType.DMA,
        ],
    )
    def segsum(y_hbm, src_hbm, dst_hbm, zeros_hbm, out0, out1,
               src_v, dst_v, rows, acc, gsem, ssem, sem):
        c = lax.axis_index("c")
        s = lax.axis_index("s")
        wid = s * NC + c
        cbase = wid * NCHUNK

        pltpu.async_copy(src_hbm.at[pl.ds(cbase, NCHUNK)], src_v, sem)
        pltpu.async_copy(dst_hbm.at[pl.ds(cbase, NCHUNK)], dst_v, sem)

        rbase = s * ROWS_PER_TILE
        pltpu.sync_copy(zeros_hbm.at[pl.ds(rbase, ROWS_PER_TILE)],
                        acc.at[pl.ds(rbase, ROWS_PER_TILE)])

        @pl.when(s == 0)
        def _():
            pltpu.sync_copy(zeros_hbm.at[pl.ds(NS * ROWS_PER_TILE, REM_ROWS)],
                            acc.at[pl.ds(NS * ROWS_PER_TILE, REM_ROWS)])

        pltpu.make_async_copy(src_hbm.at[pl.ds(cbase, NCHUNK)], src_v,
                              sem).wait()
        pltpu.make_async_copy(dst_hbm.at[pl.ds(cbase, NCHUNK)], dst_v,
                              sem).wait()
        plsc.subcore_barrier()

        for b in range(NBUF - 1):
            pltpu.async_copy(y_hbm.at[src_v.at[b]], rows[b], gsem[b])

        pltpu.make_async_copy(y_hbm.at[src_v.at[0]], rows[0], gsem[0]).wait()
        pltpu.async_copy(rows[0], acc.at[dst_v.at[0]], ssem[0], add=True)
        pltpu.async_copy(y_hbm.at[src_v.at[NBUF - 1]], rows[NBUF - 1],
                         gsem[NBUF - 1])

        @pl.loop(1, NCHUNK - NBUF + 1, step=NBUF)
        def _(j):
            for t in range(NBUF):
                i = j + t
                b = (1 + t) % NBUF
                nb = (b + NBUF - 1) % NBUF
                pltpu.make_async_copy(y_hbm.at[src_v.at[i]], rows[b],
                                      gsem[b]).wait()
                pltpu.async_copy(rows[b], acc.at[dst_v.at[i]], ssem[b],
                                 add=True)
                pltpu.make_async_copy(rows[nb], acc.at[dst_v.at[i - 1]],
                                      ssem[nb]).wait()
                pltpu.async_copy(y_hbm.at[src_v.at[i + NBUF - 1]], rows[nb],
                                 gsem[nb])

        for i in range(NCHUNK - NBUF + 1, NCHUNK):
            b = i % NBUF
            pltpu.make_async_copy(y_hbm.at[src_v.at[i]], rows[b],
                                  gsem[b]).wait()
            pltpu.async_copy(rows[b], acc.at[dst_v.at[i]], ssem[b], add=True)

        for b in range(NBUF):
            i = NCHUNK - NBUF + b
            pltpu.make_async_copy(rows[b], acc.at[dst_v.at[i]],
                                  ssem[b]).wait()
        plsc.subcore_barrier()

        @pl.when(c == 0)
        def _():
            pltpu.sync_copy(acc.at[pl.ds(rbase, ROWS_PER_TILE)],
                            out0.at[pl.ds(rbase, ROWS_PER_TILE)])

            @pl.when(s == 0)
            def _():
                pltpu.sync_copy(acc.at[pl.ds(NS * ROWS_PER_TILE, REM_ROWS)],
                                out0.at[pl.ds(NS * ROWS_PER_TILE, REM_ROWS)])

        @pl.when(c == 1)
        def _():
            pltpu.sync_copy(acc.at[pl.ds(rbase, ROWS_PER_TILE)],
                            out1.at[pl.ds(rbase, ROWS_PER_TILE)])

            @pl.when(s == 0)
            def _():
                pltpu.sync_copy(acc.at[pl.ds(NS * ROWS_PER_TILE, REM_ROWS)],
                                out1.at[pl.ds(NS * ROWS_PER_TILE, REM_ROWS)])

    return segsum


_BLK = 1000


def _mm_body(x_ref, w_ref, o_ref):
    o_ref[...] = jnp.dot(x_ref[...], w_ref[...],
                         preferred_element_type=jnp.float32)


def _tc_matmul(x, w):
    m, k = x.shape
    n = w.shape[1]
    return pl.pallas_call(
        _mm_body,
        grid=(m // _BLK,),
        in_specs=[
            pl.BlockSpec((_BLK, k), lambda i: (i, 0)),
            pl.BlockSpec((k, n), lambda i: (0, 0)),
        ],
        out_specs=pl.BlockSpec((_BLK, n), lambda i: (i, 0)),
        out_shape=jax.ShapeDtypeStruct((m, n), jnp.float32),
    )(x, w)


def _fuse_body(y_ref, p0_ref, p1_ref, b_ref, w_ref, o_ref):
    z = y_ref[...] + p0_ref[...] + p1_ref[...] + b_ref[...]
    z = jnp.maximum(z, 0.0)
    o_ref[...] = jnp.dot(z, w_ref[...], preferred_element_type=jnp.float32)


def _tc_fuse_matmul(y, p0, p1, b, w):
    m, k = y.shape
    n = w.shape[1]
    return pl.pallas_call(
        _fuse_body,
        grid=(m // _BLK,),
        in_specs=[
            pl.BlockSpec((_BLK, k), lambda i: (i, 0)),
            pl.BlockSpec((_BLK, k), lambda i: (i, 0)),
            pl.BlockSpec((_BLK, k), lambda i: (i, 0)),
            pl.BlockSpec((1, k), lambda i: (0, 0)),
            pl.BlockSpec((k, n), lambda i: (0, 0)),
        ],
        out_specs=pl.BlockSpec((_BLK, n), lambda i: (i, 0)),
        out_shape=jax.ShapeDtypeStruct((m, n), jnp.float32),
    )(y, p0, p1, b, w)


def _final_body(y_ref, q0_ref, q1_ref, b_ref, o_ref):
    o_ref[...] = y_ref[...] + q0_ref[...] + q1_ref[...] + b_ref[...]


def _tc_final(y, q0, q1, b):
    m, n = y.shape
    return pl.pallas_call(
        _final_body,
        grid=(m // _BLK,),
        in_specs=[
            pl.BlockSpec((_BLK, n), lambda i: (i, 0)),
            pl.BlockSpec((_BLK, n), lambda i: (i, 0)),
            pl.BlockSpec((_BLK, n), lambda i: (i, 0)),
            pl.BlockSpec((1, n), lambda i: (0, 0)),
        ],
        out_specs=pl.BlockSpec((_BLK, n), lambda i: (i, 0)),
        out_shape=jax.ShapeDtypeStruct((m, n), jnp.float32),
    )(y, q0, q1, b)


def kernel(features, edge_index, W1, b1, W2, b2):
    src1 = edge_index[0].reshape(E // _CHUNK[D1], _CHUNK[D1])
    dst1 = edge_index[1].reshape(E // _CHUNK[D1], _CHUNK[D1])
    src2 = edge_index[0].reshape(E // _CHUNK[D2], _CHUNK[D2])
    dst2 = edge_index[1].reshape(E // _CHUNK[D2], _CHUNK[D2])
    z1 = jnp.zeros((N, D1), jnp.float32)
    z2 = jnp.zeros((N, D2), jnp.float32)
    b1r = b1.reshape(1, D1)
    b2r = b2.reshape(1, D2)

    y1 = _tc_matmul(features, W1)
    p0, p1 = _make_sc_segsum(D1)(y1, src1, dst1, z1)
    y2 = _tc_fuse_matmul(y1, p0, p1, b1r, W2)
    q0, q1 = _make_sc_segsum(D2)(y2, src2, dst2, z2)
    return _tc_final(y2, q0, q1, b2r)


# speedup vs baseline: 1.0159x; 1.0159x over previous
import functools

import jax
import jax.numpy as jnp
from jax import lax
from jax.experimental import pallas as pl
from jax.experimental.pallas import tpu as pltpu
from jax.experimental.pallas import tpu_sc as plsc

N = 10000
E = 320000
D1 = 128
D2 = 40

NC = 2
NS = 16
NW = NC * NS
E_PER_W = E // NW
ROWS_PER_TILE = 624
REM_ROWS = N - NS * ROWS_PER_TILE

_CHUNK = {128: 40, 40: 40}
_NBUF = {128: 5, 40: 10}


@functools.lru_cache(maxsize=None)
def _make_sc_segsum(D):
    CHUNK = _CHUNK[D]
    NBUF = _NBUF[D]
    NCHUNK = E_PER_W // CHUNK
    mesh = plsc.VectorSubcoreMesh(core_axis_name="c", subcore_axis_name="s",
                                  num_cores=NC, num_subcores=NS)

    @functools.partial(
        pl.kernel,
        out_type=(
            jax.ShapeDtypeStruct((N, D), jnp.float32),
            jax.ShapeDtypeStruct((N, D), jnp.float32),
        ),
        mesh=mesh,
        compiler_params=pltpu.CompilerParams(use_tc_tiling_on_sc=False),
        scratch_types=[
            pltpu.VMEM((NCHUNK, CHUNK), jnp.int32),
            pltpu.VMEM((NCHUNK, CHUNK), jnp.int32),
            [pltpu.VMEM((CHUNK, D), jnp.float32) for _ in range(NBUF)],
            pltpu.VMEM_SHARED((N, D), jnp.float32),
            [pltpu.SemaphoreType.DMA for _ in range(NBUF)],
            [pltpu.SemaphoreType.DMA for _ in range(NBUF)],
            pltpu.SemaphoreType.DMA,
        ],
    )
    def segsum(y_hbm, src_hbm, dst_hbm, zeros_hbm, out0, out1,
               src_v, dst_v, rows, acc, gsem, ssem, sem):
        c = lax.axis_index("c")
        s = lax.axis_index("s")
        wid = s * NC + c
        cbase = wid * NCHUNK

        pltpu.async_copy(src_hbm.at[pl.ds(cbase, NCHUNK)], src_v, sem)
        pltpu.async_copy(dst_hbm.at[pl.ds(cbase, NCHUNK)], dst_v, sem)

        rbase = s * ROWS_PER_TILE
        pltpu.sync_copy(zeros_hbm.at[pl.ds(rbase, ROWS_PER_TILE)],
                        acc.at[pl.ds(rbase, ROWS_PER_TILE)])

        @pl.when(s == 0)
        def _():
            pltpu.sync_copy(zeros_hbm.at[pl.ds(NS * ROWS_PER_TILE, REM_ROWS)],
                            acc.at[pl.ds(NS * ROWS_PER_TILE, REM_ROWS)])

        pltpu.make_async_copy(src_hbm.at[pl.ds(cbase, NCHUNK)], src_v,
                              sem).wait()
        pltpu.make_async_copy(dst_hbm.at[pl.ds(cbase, NCHUNK)], dst_v,
                              sem).wait()
        plsc.subcore_barrier()

        for b in range(NBUF - 1):
            pltpu.async_copy(y_hbm.at[src_v.at[b]], rows[b], gsem[b])

        pltpu.make_async_copy(y_hbm.at[src_v.at[0]], rows[0], gsem[0]).wait()
        pltpu.async_copy(rows[0], acc.at[dst_v.at[0]], ssem[0], add=True)
        pltpu.async_copy(y_hbm.at[src_v.at[NBUF - 1]], rows[NBUF - 1],
                         gsem[NBUF - 1])

        @pl.loop(1, NCHUNK - NBUF + 1, step=NBUF)
        def _(j):
            for t in range(NBUF):
                i = j + t
                b = (1 + t) % NBUF
                nb = (b + NBUF - 1) % NBUF
                pltpu.make_async_copy(y_hbm.at[src_v.at[i]], rows[b],
                                      gsem[b]).wait()
                pltpu.async_copy(rows[b], acc.at[dst_v.at[i]], ssem[b],
                                 add=True)
                pltpu.make_async_copy(rows[nb], acc.at[dst_v.at[i - 1]],
                                      ssem[nb]).wait()
                pltpu.async_copy(y_hbm.at[src_v.at[i + NBUF - 1]], rows[nb],
                                 gsem[nb])

        for i in range(NCHUNK - NBUF + 1, NCHUNK):
            b = i % NBUF
            pltpu.make_async_copy(y_hbm.at[src_v.at[i]], rows[b],
                                  gsem[b]).wait()
            pltpu.async_copy(rows[b], acc.at[dst_v.at[i]], ssem[b], add=True)

        for b in range(NBUF):
            i = NCHUNK - NBUF + b
            pltpu.make_async_copy(rows[b], acc.at[dst_v.at[i]],
                                  ssem[b]).wait()
        plsc.subcore_barrier()

        @pl.when(c == 0)
        def _():
            pltpu.sync_copy(acc.at[pl.ds(rbase, ROWS_PER_TILE)],
                            out0.at[pl.ds(rbase, ROWS_PER_TILE)])

            @pl.when(s == 0)
            def _():
                pltpu.sync_copy(acc.at[pl.ds(NS * ROWS_PER_TILE, REM_ROWS)],
                                out0.at[pl.ds(NS * ROWS_PER_TILE, REM_ROWS)])

        @pl.when(c == 1)
        def _():
            pltpu.sync_copy(acc.at[pl.ds(rbase, ROWS_PER_TILE)],
                            out1.at[pl.ds(rbase, ROWS_PER_TILE)])

            @pl.when(s == 0)
            def _():
                pltpu.sync_copy(acc.at[pl.ds(NS * ROWS_PER_TILE, REM_ROWS)],
                                out1.at[pl.ds(NS * ROWS_PER_TILE, REM_ROWS)])

    return segsum


_BLK = 1000


def _mm_body(x_ref, w_ref, o_ref):
    o_ref[...] = jnp.dot(x_ref[...], w_ref[...],
                         preferred_element_type=jnp.float32)


def _tc_matmul(x, w):
    m, k = x.shape
    n = w.shape[1]
    return pl.pallas_call(
        _mm_body,
        grid=(m // _BLK,),
        in_specs=[
            pl.BlockSpec((_BLK, k), lambda i: (i, 0)),
            pl.BlockSpec((k, n), lambda i: (0, 0)),
        ],
        out_specs=pl.BlockSpec((_BLK, n), lambda i: (i, 0)),
        out_shape=jax.ShapeDtypeStruct((m, n), jnp.float32),
    )(x, w)


def _fuse_body(y_ref, p0_ref, p1_ref, b_ref, w_ref, o_ref):
    z = y_ref[...] + p0_ref[...] + p1_ref[...] + b_ref[...]
    z = jnp.maximum(z, 0.0)
    o_ref[...] = jnp.dot(z, w_ref[...], preferred_element_type=jnp.float32)


def _tc_fuse_matmul(y, p0, p1, b, w):
    m, k = y.shape
    n = w.shape[1]
    return pl.pallas_call(
        _fuse_body,
        grid=(m // _BLK,),
        in_specs=[
            pl.BlockSpec((_BLK, k), lambda i: (i, 0)),
            pl.BlockSpec((_BLK, k), lambda i: (i, 0)),
            pl.BlockSpec((_BLK, k), lambda i: (i, 0)),
            pl.BlockSpec((1, k), lambda i: (0, 0)),
            pl.BlockSpec((k, n), lambda i: (0, 0)),
        ],
        out_specs=pl.BlockSpec((_BLK, n), lambda i: (i, 0)),
        out_shape=jax.ShapeDtypeStruct((m, n), jnp.float32),
    )(y, p0, p1, b, w)


def _final_body(y_ref, q0_ref, q1_ref, b_ref, o_ref):
    o_ref[...] = y_ref[...] + q0_ref[...] + q1_ref[...] + b_ref[...]


def _tc_final(y, q0, q1, b):
    m, n = y.shape
    return pl.pallas_call(
        _final_body,
        grid=(m // _BLK,),
        in_specs=[
            pl.BlockSpec((_BLK, n), lambda i: (i, 0)),
            pl.BlockSpec((_BLK, n), lambda i: (i, 0)),
            pl.BlockSpec((_BLK, n), lambda i: (i, 0)),
            pl.BlockSpec((1, n), lambda i: (0, 0)),
        ],
        out_specs=pl.BlockSpec((_BLK, n), lambda i: (i, 0)),
        out_shape=jax.ShapeDtypeStruct((m, n), jnp.float32),
    )(y, q0, q1, b)


def kernel(features, edge_index, W1, b1, W2, b2):
    src1 = edge_index[0].reshape(E // _CHUNK[D1], _CHUNK[D1])
    dst1 = edge_index[1].reshape(E // _CHUNK[D1], _CHUNK[D1])
    src2, dst2 = src1, dst1
    z1 = jnp.zeros((N, D1), jnp.float32)
    z2 = jnp.zeros((N, D2), jnp.float32)
    b1r = b1.reshape(1, D1)
    b2r = b2.reshape(1, D2)

    y1 = _tc_matmul(features, W1)
    p0, p1 = _make_sc_segsum(D1)(y1, src1, dst1, z1)
    y2 = _tc_fuse_matmul(y1, p0, p1, b1r, W2)
    q0, q1 = _make_sc_segsum(D2)(y2, src2, dst2, z2)
    return _tc_final(y2, q0, q1, b2r)


# speedup vs baseline: 1.1113x; 1.0940x over previous
import functools

import jax
import jax.numpy as jnp
from jax import lax
from jax.experimental import pallas as pl
from jax.experimental.pallas import tpu as pltpu
from jax.experimental.pallas import tpu_sc as plsc

N = 10000
E = 320000
D1 = 128
D2 = 40

NC = 2
NS = 16
NW = NC * NS
E_PER_W = E // NW
ROWS_PER_TILE = 624
REM_ROWS = N - NS * ROWS_PER_TILE

_CHUNK = {128: 40, 40: 40}
_NBUF = {128: 5, 40: 10}


@functools.lru_cache(maxsize=None)
def _make_sc_segsum(D):
    CHUNK = _CHUNK[D]
    NBUF = _NBUF[D]
    NCHUNK = E_PER_W // CHUNK
    mesh = plsc.VectorSubcoreMesh(core_axis_name="c", subcore_axis_name="s",
                                  num_cores=NC, num_subcores=NS)

    @functools.partial(
        pl.kernel,
        out_type=(
            jax.ShapeDtypeStruct((N, D), jnp.float32),
            jax.ShapeDtypeStruct((N, D), jnp.float32),
        ),
        mesh=mesh,
        compiler_params=pltpu.CompilerParams(use_tc_tiling_on_sc=False),
        scratch_types=[
            pltpu.VMEM((NCHUNK, CHUNK), jnp.int32),
            pltpu.VMEM((NCHUNK, CHUNK), jnp.int32),
            [pltpu.VMEM((CHUNK, D), jnp.float32) for _ in range(NBUF)],
            pltpu.VMEM_SHARED((N, D), jnp.float32),
            [pltpu.SemaphoreType.DMA for _ in range(NBUF)],
            [pltpu.SemaphoreType.DMA for _ in range(NBUF)],
            pltpu.SemaphoreType.DMA,
        ],
    )
    def segsum(y_hbm, ei_hbm, zeros_hbm, out0, out1,
               src_v, dst_v, rows, acc, gsem, ssem, sem):
        c = lax.axis_index("c")
        s = lax.axis_index("s")
        wid = s * NC + c
        cbase = wid * NCHUNK

        pltpu.async_copy(ei_hbm.at[0, pl.ds(cbase, NCHUNK)], src_v, sem)
        pltpu.async_copy(ei_hbm.at[1, pl.ds(cbase, NCHUNK)], dst_v, sem)

        rbase = s * ROWS_PER_TILE
        pltpu.sync_copy(zeros_hbm.at[pl.ds(rbase, ROWS_PER_TILE)],
                        acc.at[pl.ds(rbase, ROWS_PER_TILE)])

        @pl.when(s == 0)
        def _():
            pltpu.sync_copy(zeros_hbm.at[pl.ds(NS * ROWS_PER_TILE, REM_ROWS)],
                            acc.at[pl.ds(NS * ROWS_PER_TILE, REM_ROWS)])

        pltpu.make_async_copy(ei_hbm.at[0, pl.ds(cbase, NCHUNK)], src_v,
                              sem).wait()
        pltpu.make_async_copy(ei_hbm.at[1, pl.ds(cbase, NCHUNK)], dst_v,
                              sem).wait()
        plsc.subcore_barrier()

        for b in range(NBUF - 1):
            pltpu.async_copy(y_hbm.at[src_v.at[b]], rows[b], gsem[b])

        pltpu.make_async_copy(y_hbm.at[src_v.at[0]], rows[0], gsem[0]).wait()
        pltpu.async_copy(rows[0], acc.at[dst_v.at[0]], ssem[0], add=True)
        pltpu.async_copy(y_hbm.at[src_v.at[NBUF - 1]], rows[NBUF - 1],
                         gsem[NBUF - 1])

        @pl.loop(1, NCHUNK - NBUF + 1, step=NBUF)
        def _(j):
            for t in range(NBUF):
                i = j + t
                b = (1 + t) % NBUF
                nb = (b + NBUF - 1) % NBUF
                pltpu.make_async_copy(y_hbm.at[src_v.at[i]], rows[b],
                                      gsem[b]).wait()
                pltpu.async_copy(rows[b], acc.at[dst_v.at[i]], ssem[b],
                                 add=True)
                pltpu.make_async_copy(rows[nb], acc.at[dst_v.at[i - 1]],
                                      ssem[nb]).wait()
                pltpu.async_copy(y_hbm.at[src_v.at[i + NBUF - 1]], rows[nb],
                                 gsem[nb])

        for i in range(NCHUNK - NBUF + 1, NCHUNK):
            b = i % NBUF
            pltpu.make_async_copy(y_hbm.at[src_v.at[i]], rows[b],
                                  gsem[b]).wait()
            pltpu.async_copy(rows[b], acc.at[dst_v.at[i]], ssem[b], add=True)

        for b in range(NBUF):
            i = NCHUNK - NBUF + b
            pltpu.make_async_copy(rows[b], acc.at[dst_v.at[i]],
                                  ssem[b]).wait()
        plsc.subcore_barrier()

        @pl.when(c == 0)
        def _():
            pltpu.sync_copy(acc.at[pl.ds(rbase, ROWS_PER_TILE)],
                            out0.at[pl.ds(rbase, ROWS_PER_TILE)])

            @pl.when(s == 0)
            def _():
                pltpu.sync_copy(acc.at[pl.ds(NS * ROWS_PER_TILE, REM_ROWS)],
                                out0.at[pl.ds(NS * ROWS_PER_TILE, REM_ROWS)])

        @pl.when(c == 1)
        def _():
            pltpu.sync_copy(acc.at[pl.ds(rbase, ROWS_PER_TILE)],
                            out1.at[pl.ds(rbase, ROWS_PER_TILE)])

            @pl.when(s == 0)
            def _():
                pltpu.sync_copy(acc.at[pl.ds(NS * ROWS_PER_TILE, REM_ROWS)],
                                out1.at[pl.ds(NS * ROWS_PER_TILE, REM_ROWS)])

    return segsum


_CB_ROWS = 312


@functools.lru_cache(maxsize=None)
def _make_sc_combine():
    mesh = plsc.VectorSubcoreMesh(core_axis_name="c", subcore_axis_name="s",
                                  num_cores=NC, num_subcores=NS)

    @functools.partial(
        pl.kernel,
        out_type=jax.ShapeDtypeStruct((N, D2), jnp.float32),
        mesh=mesh,
        compiler_params=pltpu.CompilerParams(use_tc_tiling_on_sc=False),
        scratch_types=[
            pltpu.VMEM((_CB_ROWS, D2), jnp.float32),
            pltpu.VMEM((_CB_ROWS, D2), jnp.float32),
            pltpu.VMEM((_CB_ROWS, D2), jnp.float32),
            pltpu.VMEM((D2,), jnp.float32),
            [pltpu.SemaphoreType.DMA for _ in range(3)],
        ],
    )
    def combine(y_hbm, q0_hbm, q1_hbm, b_hbm, out_hbm, yv, q0v, q1v, bv,
                sems):
        c = lax.axis_index("c")
        s = lax.axis_index("s")
        wid = s * NC + c

        pltpu.sync_copy(b_hbm, bv)
        segs = [(0, bv[pl.ds(0, 16)]), (16, bv[pl.ds(16, 16)]),
                (24, bv[pl.ds(24, 16)])]

        def do_block(r0, nrows):
            pltpu.async_copy(y_hbm.at[pl.ds(r0, nrows)], yv.at[pl.ds(0, nrows)], sems[0])
            pltpu.async_copy(q0_hbm.at[pl.ds(r0, nrows)], q0v.at[pl.ds(0, nrows)], sems[1])
            pltpu.async_copy(q1_hbm.at[pl.ds(r0, nrows)], q1v.at[pl.ds(0, nrows)], sems[2])
            pltpu.make_async_copy(y_hbm.at[pl.ds(r0, nrows)], yv.at[pl.ds(0, nrows)], sems[0]).wait()
            pltpu.make_async_copy(q0_hbm.at[pl.ds(r0, nrows)], q0v.at[pl.ds(0, nrows)], sems[1]).wait()
            pltpu.make_async_copy(q1_hbm.at[pl.ds(r0, nrows)], q1v.at[pl.ds(0, nrows)], sems[2]).wait()

            def row(i, carry):
                vals = [(off,
                         yv[i, pl.ds(off, 16)] + q0v[i, pl.ds(off, 16)]
                         + q1v[i, pl.ds(off, 16)] + bseg)
                        for off, bseg in segs]
                for off, v in vals:
                    yv[i, pl.ds(off, 16)] = v
                return carry

            lax.fori_loop(0, nrows, row, 0)
            pltpu.sync_copy(yv.at[pl.ds(0, nrows)],
                            out_hbm.at[pl.ds(r0, nrows)])

        do_block(wid * _CB_ROWS, _CB_ROWS)

        @pl.when(wid == 0)
        def _():
            do_block(NW * _CB_ROWS, N - NW * _CB_ROWS)

    return combine


_BLK = 1000


def _mm_body(x_ref, w_ref, o_ref):
    o_ref[...] = jnp.dot(x_ref[...], w_ref[...],
                         preferred_element_type=jnp.float32)


def _tc_matmul(x, w):
    m, k = x.shape
    n = w.shape[1]
    return pl.pallas_call(
        _mm_body,
        grid=(m // _BLK,),
        in_specs=[
            pl.BlockSpec((_BLK, k), lambda i: (i, 0)),
            pl.BlockSpec((k, n), lambda i: (0, 0)),
        ],
        out_specs=pl.BlockSpec((_BLK, n), lambda i: (i, 0)),
        out_shape=jax.ShapeDtypeStruct((m, n), jnp.float32),
    )(x, w)


def _fuse_body(y_ref, p0_ref, p1_ref, b_ref, w_ref, o_ref):
    z = y_ref[...] + p0_ref[...] + p1_ref[...] + b_ref[...]
    z = jnp.maximum(z, 0.0)
    o_ref[...] = jnp.dot(z, w_ref[...], preferred_element_type=jnp.float32)


def _tc_fuse_matmul(y, p0, p1, b, w):
    m, k = y.shape
    n = w.shape[1]
    return pl.pallas_call(
        _fuse_body,
        grid=(m // _BLK,),
        in_specs=[
            pl.BlockSpec((_BLK, k), lambda i: (i, 0)),
            pl.BlockSpec((_BLK, k), lambda i: (i, 0)),
            pl.BlockSpec((_BLK, k), lambda i: (i, 0)),
            pl.BlockSpec((1, k), lambda i: (0, 0)),
            pl.BlockSpec((k, n), lambda i: (0, 0)),
        ],
        out_specs=pl.BlockSpec((_BLK, n), lambda i: (i, 0)),
        out_shape=jax.ShapeDtypeStruct((m, n), jnp.float32),
    )(y, p0, p1, b, w)


def _final_body(y_ref, q0_ref, q1_ref, b_ref, o_ref):
    o_ref[...] = y_ref[...] + q0_ref[...] + q1_ref[...] + b_ref[...]


def _tc_final(y, q0, q1, b):
    m, n = y.shape
    return pl.pallas_call(
        _final_body,
        grid=(m // _BLK,),
        in_specs=[
            pl.BlockSpec((_BLK, n), lambda i: (i, 0)),
            pl.BlockSpec((_BLK, n), lambda i: (i, 0)),
            pl.BlockSpec((_BLK, n), lambda i: (i, 0)),
            pl.BlockSpec((1, n), lambda i: (0, 0)),
        ],
        out_specs=pl.BlockSpec((_BLK, n), lambda i: (i, 0)),
        out_shape=jax.ShapeDtypeStruct((m, n), jnp.float32),
    )(y, q0, q1, b)


def kernel(features, edge_index, W1, b1, W2, b2):
    ei3 = edge_index.reshape(2, E // _CHUNK[D1], _CHUNK[D1])
    z1 = jnp.zeros((N, D1), jnp.float32)
    z2 = jnp.zeros((N, D2), jnp.float32)
    b1r = b1.reshape(1, D1)

    y1 = _tc_matmul(features, W1)
    p0, p1 = _make_sc_segsum(D1)(y1, ei3, z1)
    y2 = _tc_fuse_matmul(y1, p0, p1, b1r, W2)
    q0, q1 = _make_sc_segsum(D2)(y2, ei3, z2)
    return _make_sc_combine()(y2, q0, q1, b2)


# speedup vs baseline: 1.1407x; 1.0264x over previous
import functools

import jax
import jax.numpy as jnp
import numpy as np
from jax import lax
from jax.experimental import pallas as pl
from jax.experimental.pallas import tpu as pltpu
from jax.experimental.pallas import tpu_sc as plsc

N = 10000
E = 320000
D1 = 128
D2 = 40

NC = 2
NS = 16
NW = NC * NS
E_PER_W = E // NW
ROWS_PER_TILE = 624
REM_ROWS = N - NS * ROWS_PER_TILE

_CHUNK = {128: 40, 40: 40}
_NBUF = {128: 5, 40: 10}


@functools.lru_cache(maxsize=None)
def _make_sc_segsum(D):
    CHUNK = _CHUNK[D]
    NBUF = _NBUF[D]
    NCHUNK = E_PER_W // CHUNK
    mesh = plsc.VectorSubcoreMesh(core_axis_name="c", subcore_axis_name="s",
                                  num_cores=NC, num_subcores=NS)

    @functools.partial(
        pl.kernel,
        out_type=(
            jax.ShapeDtypeStruct((N, D), jnp.float32),
            jax.ShapeDtypeStruct((N, D), jnp.float32),
        ),
        mesh=mesh,
        compiler_params=pltpu.CompilerParams(use_tc_tiling_on_sc=False),
        scratch_types=[
            pltpu.VMEM((NCHUNK, CHUNK), jnp.int32),
            pltpu.VMEM((NCHUNK, CHUNK), jnp.int32),
            [pltpu.VMEM((CHUNK, D), jnp.float32) for _ in range(NBUF)],
            pltpu.VMEM_SHARED((N, D), jnp.float32),
            [pltpu.SemaphoreType.DMA for _ in range(NBUF)],
            [pltpu.SemaphoreType.DMA for _ in range(NBUF)],
            pltpu.SemaphoreType.DMA,
        ],
    )
    def segsum(y_hbm, ei_hbm, zeros_hbm, out0, out1,
               src_v, dst_v, rows, acc, gsem, ssem, sem):
        c = lax.axis_index("c")
        s = lax.axis_index("s")
        wid = s * NC + c
        cbase = wid * NCHUNK

        pltpu.async_copy(ei_hbm.at[0, pl.ds(cbase, NCHUNK)], src_v, sem)
        pltpu.async_copy(ei_hbm.at[1, pl.ds(cbase, NCHUNK)], dst_v, sem)

        rbase = s * ROWS_PER_TILE
        pltpu.sync_copy(zeros_hbm.at[pl.ds(rbase, ROWS_PER_TILE)],
                        acc.at[pl.ds(rbase, ROWS_PER_TILE)])

        @pl.when(s == 0)
        def _():
            pltpu.sync_copy(zeros_hbm.at[pl.ds(NS * ROWS_PER_TILE, REM_ROWS)],
                            acc.at[pl.ds(NS * ROWS_PER_TILE, REM_ROWS)])

        pltpu.make_async_copy(ei_hbm.at[0, pl.ds(cbase, NCHUNK)], src_v,
                              sem).wait()
        pltpu.make_async_copy(ei_hbm.at[1, pl.ds(cbase, NCHUNK)], dst_v,
                              sem).wait()
        plsc.subcore_barrier()

        for b in range(NBUF - 1):
            pltpu.async_copy(y_hbm.at[src_v.at[b]], rows[b], gsem[b])

        pltpu.make_async_copy(y_hbm.at[src_v.at[0]], rows[0], gsem[0]).wait()
        pltpu.async_copy(rows[0], acc.at[dst_v.at[0]], ssem[0], add=True)
        pltpu.async_copy(y_hbm.at[src_v.at[NBUF - 1]], rows[NBUF - 1],
                         gsem[NBUF - 1])

        @pl.loop(1, NCHUNK - NBUF + 1, step=NBUF)
        def _(j):
            for t in range(NBUF):
                i = j + t
                b = (1 + t) % NBUF
                nb = (b + NBUF - 1) % NBUF
                pltpu.make_async_copy(y_hbm.at[src_v.at[i]], rows[b],
                                      gsem[b]).wait()
                pltpu.async_copy(rows[b], acc.at[dst_v.at[i]], ssem[b],
                                 add=True)
                pltpu.make_async_copy(rows[nb], acc.at[dst_v.at[i - 1]],
                                      ssem[nb]).wait()
                pltpu.async_copy(y_hbm.at[src_v.at[i + NBUF - 1]], rows[nb],
                                 gsem[nb])

        for i in range(NCHUNK - NBUF + 1, NCHUNK):
            b = i % NBUF
            pltpu.make_async_copy(y_hbm.at[src_v.at[i]], rows[b],
                                  gsem[b]).wait()
            pltpu.async_copy(rows[b], acc.at[dst_v.at[i]], ssem[b], add=True)

        for b in range(NBUF):
            i = NCHUNK - NBUF + b
            pltpu.make_async_copy(rows[b], acc.at[dst_v.at[i]],
                                  ssem[b]).wait()
        plsc.subcore_barrier()

        @pl.when(c == 0)
        def _():
            pltpu.sync_copy(acc.at[pl.ds(rbase, ROWS_PER_TILE)],
                            out0.at[pl.ds(rbase, ROWS_PER_TILE)])

            @pl.when(s == 0)
            def _():
                pltpu.sync_copy(acc.at[pl.ds(NS * ROWS_PER_TILE, REM_ROWS)],
                                out0.at[pl.ds(NS * ROWS_PER_TILE, REM_ROWS)])

        @pl.when(c == 1)
        def _():
            pltpu.sync_copy(acc.at[pl.ds(rbase, ROWS_PER_TILE)],
                            out1.at[pl.ds(rbase, ROWS_PER_TILE)])

            @pl.when(s == 0)
            def _():
                pltpu.sync_copy(acc.at[pl.ds(NS * ROWS_PER_TILE, REM_ROWS)],
                                out1.at[pl.ds(NS * ROWS_PER_TILE, REM_ROWS)])

    return segsum


_CB_ROWS = 312


@functools.lru_cache(maxsize=None)
def _make_sc_combine():
    mesh = plsc.VectorSubcoreMesh(core_axis_name="c", subcore_axis_name="s",
                                  num_cores=NC, num_subcores=NS)

    @functools.partial(
        pl.kernel,
        out_type=jax.ShapeDtypeStruct((N, D2), jnp.float32),
        mesh=mesh,
        compiler_params=pltpu.CompilerParams(use_tc_tiling_on_sc=False),
        scratch_types=[
            pltpu.VMEM((_CB_ROWS, D2), jnp.float32),
            pltpu.VMEM((_CB_ROWS, D2), jnp.float32),
            pltpu.VMEM((_CB_ROWS, D2), jnp.float32),
            pltpu.VMEM((D2,), jnp.float32),
            [pltpu.SemaphoreType.DMA for _ in range(3)],
        ],
    )
    def combine(y_hbm, q0_hbm, q1_hbm, b_hbm, out_hbm, yv, q0v, q1v, bv,
                sems):
        c = lax.axis_index("c")
        s = lax.axis_index("s")
        wid = s * NC + c

        pltpu.sync_copy(b_hbm, bv)
        segs = [(0, bv[pl.ds(0, 16)]), (16, bv[pl.ds(16, 16)]),
                (24, bv[pl.ds(24, 16)])]

        def do_block(r0, nrows):
            pltpu.async_copy(y_hbm.at[pl.ds(r0, nrows)], yv.at[pl.ds(0, nrows)], sems[0])
            pltpu.async_copy(q0_hbm.at[pl.ds(r0, nrows)], q0v.at[pl.ds(0, nrows)], sems[1])
            pltpu.async_copy(q1_hbm.at[pl.ds(r0, nrows)], q1v.at[pl.ds(0, nrows)], sems[2])
            pltpu.make_async_copy(y_hbm.at[pl.ds(r0, nrows)], yv.at[pl.ds(0, nrows)], sems[0]).wait()
            pltpu.make_async_copy(q0_hbm.at[pl.ds(r0, nrows)], q0v.at[pl.ds(0, nrows)], sems[1]).wait()
            pltpu.make_async_copy(q1_hbm.at[pl.ds(r0, nrows)], q1v.at[pl.ds(0, nrows)], sems[2]).wait()

            def row(i, carry):
                vals = [(off,
                         yv[i, pl.ds(off, 16)] + q0v[i, pl.ds(off, 16)]
                         + q1v[i, pl.ds(off, 16)] + bseg)
                        for off, bseg in segs]
                for off, v in vals:
                    yv[i, pl.ds(off, 16)] = v
                return carry

            lax.fori_loop(0, nrows, row, 0)
            pltpu.sync_copy(yv.at[pl.ds(0, nrows)],
                            out_hbm.at[pl.ds(r0, nrows)])

        do_block(wid * _CB_ROWS, _CB_ROWS)

        @pl.when(wid == 0)
        def _():
            do_block(NW * _CB_ROWS, N - NW * _CB_ROWS)

    return combine


_BLK = 2000


def _mm_body(x_ref, w_ref, o_ref):
    o_ref[...] = jnp.dot(x_ref[...], w_ref[...],
                         preferred_element_type=jnp.float32)


def _tc_matmul(x, w):
    m, k = x.shape
    n = w.shape[1]
    return pl.pallas_call(
        _mm_body,
        grid=(m // _BLK,),
        in_specs=[
            pl.BlockSpec((_BLK, k), lambda i: (i, 0)),
            pl.BlockSpec((k, n), lambda i: (0, 0)),
        ],
        out_specs=pl.BlockSpec((_BLK, n), lambda i: (i, 0)),
        out_shape=jax.ShapeDtypeStruct((m, n), jnp.float32),
    )(x, w)


def _fuse_body(y_ref, p0_ref, p1_ref, b_ref, w_ref, o_ref):
    z = y_ref[...] + p0_ref[...] + p1_ref[...] + b_ref[...]
    z = jnp.maximum(z, 0.0)
    o_ref[...] = jnp.dot(z, w_ref[...], preferred_element_type=jnp.float32)


def _tc_fuse_matmul(y, p0, p1, b, w):
    m, k = y.shape
    n = w.shape[1]
    return pl.pallas_call(
        _fuse_body,
        grid=(m // _BLK,),
        in_specs=[
            pl.BlockSpec((_BLK, k), lambda i: (i, 0)),
            pl.BlockSpec((_BLK, k), lambda i: (i, 0)),
            pl.BlockSpec((_BLK, k), lambda i: (i, 0)),
            pl.BlockSpec((1, k), lambda i: (0, 0)),
            pl.BlockSpec((k, n), lambda i: (0, 0)),
        ],
        out_specs=pl.BlockSpec((_BLK, n), lambda i: (i, 0)),
        out_shape=jax.ShapeDtypeStruct((m, n), jnp.float32),
    )(y, p0, p1, b, w)


def _final_body(y_ref, q0_ref, q1_ref, b_ref, o_ref):
    o_ref[...] = y_ref[...] + q0_ref[...] + q1_ref[...] + b_ref[...]


def _tc_final(y, q0, q1, b):
    m, n = y.shape
    return pl.pallas_call(
        _final_body,
        grid=(m // _BLK,),
        in_specs=[
            pl.BlockSpec((_BLK, n), lambda i: (i, 0)),
            pl.BlockSpec((_BLK, n), lambda i: (i, 0)),
            pl.BlockSpec((_BLK, n), lambda i: (i, 0)),
            pl.BlockSpec((1, n), lambda i: (0, 0)),
        ],
        out_specs=pl.BlockSpec((_BLK, n), lambda i: (i, 0)),
        out_shape=jax.ShapeDtypeStruct((m, n), jnp.float32),
    )(y, q0, q1, b)


def kernel(features, edge_index, W1, b1, W2, b2):
    ei3 = edge_index.reshape(2, E // _CHUNK[D1], _CHUNK[D1])
    z1 = jnp.asarray(np.zeros((N, D1), np.float32))
    z2 = jnp.asarray(np.zeros((N, D2), np.float32))
    b1r = b1.reshape(1, D1)

    y1 = _tc_matmul(features, W1)
    p0, p1 = _make_sc_segsum(D1)(y1, ei3, z1)
    y2 = _tc_fuse_matmul(y1, p0, p1, b1r, W2)
    q0, q1 = _make_sc_segsum(D2)(y2, ei3, z2)
    return _make_sc_combine()(y2, q0, q1, b2)


# speedup vs baseline: 1.1766x; 1.0315x over previous
import functools

import jax
import jax.numpy as jnp
import numpy as np
from jax import lax
from jax.experimental import pallas as pl
from jax.experimental.pallas import tpu as pltpu
from jax.experimental.pallas import tpu_sc as plsc

N = 10000
E = 320000
D1 = 128
D2 = 40

NC = 2
NS = 16
NW = NC * NS
E_PER_W = E // NW
ROWS_PER_TILE = 624
REM_ROWS = N - NS * ROWS_PER_TILE

_CHUNK = {128: 40, 40: 40}
_NBUF = {128: 5, 40: 10}


@functools.lru_cache(maxsize=None)
def _make_sc_segsum(D):
    CHUNK = _CHUNK[D]
    NBUF = _NBUF[D]
    NCHUNK = E_PER_W // CHUNK
    mesh = plsc.VectorSubcoreMesh(core_axis_name="c", subcore_axis_name="s",
                                  num_cores=NC, num_subcores=NS)

    @functools.partial(
        pl.kernel,
        out_type=(
            jax.ShapeDtypeStruct((N, D), jnp.float32),
            jax.ShapeDtypeStruct((N, D), jnp.float32),
        ),
        mesh=mesh,
        compiler_params=pltpu.CompilerParams(use_tc_tiling_on_sc=False),
        scratch_types=[
            pltpu.VMEM((NCHUNK, CHUNK), jnp.int32),
            pltpu.VMEM((NCHUNK, CHUNK), jnp.int32),
            [pltpu.VMEM((CHUNK, D), jnp.float32) for _ in range(NBUF)],
            pltpu.VMEM_SHARED((N, D), jnp.float32),
            [pltpu.SemaphoreType.DMA for _ in range(NBUF)],
            [pltpu.SemaphoreType.DMA for _ in range(NBUF)],
            pltpu.SemaphoreType.DMA,
        ],
    )
    def segsum(y_hbm, ei_hbm, zeros_hbm, out0, out1,
               src_v, dst_v, rows, acc, gsem, ssem, sem):
        c = lax.axis_index("c")
        s = lax.axis_index("s")
        wid = s * NC + c
        cbase = wid * NCHUNK

        pltpu.async_copy(ei_hbm.at[0, pl.ds(cbase, NCHUNK)], src_v, sem)
        pltpu.async_copy(ei_hbm.at[1, pl.ds(cbase, NCHUNK)], dst_v, sem)

        rbase = s * ROWS_PER_TILE
        pltpu.sync_copy(zeros_hbm.at[pl.ds(rbase, ROWS_PER_TILE)],
                        acc.at[pl.ds(rbase, ROWS_PER_TILE)])

        @pl.when(s == 0)
        def _():
            pltpu.sync_copy(zeros_hbm.at[pl.ds(NS * ROWS_PER_TILE, REM_ROWS)],
                            acc.at[pl.ds(NS * ROWS_PER_TILE, REM_ROWS)])

        pltpu.make_async_copy(ei_hbm.at[0, pl.ds(cbase, NCHUNK)], src_v,
                              sem).wait()
        pltpu.make_async_copy(ei_hbm.at[1, pl.ds(cbase, NCHUNK)], dst_v,
                              sem).wait()
        plsc.subcore_barrier()

        for b in range(NBUF - 1):
            pltpu.async_copy(y_hbm.at[src_v.at[b]], rows[b], gsem[b])

        pltpu.make_async_copy(y_hbm.at[src_v.at[0]], rows[0], gsem[0]).wait()
        pltpu.async_copy(rows[0], acc.at[dst_v.at[0]], ssem[0], add=True)
        pltpu.async_copy(y_hbm.at[src_v.at[NBUF - 1]], rows[NBUF - 1],
                         gsem[NBUF - 1])

        @pl.loop(1, NCHUNK - NBUF + 1, step=NBUF)
        def _(j):
            for t in range(NBUF):
                i = j + t
                b = (1 + t) % NBUF
                nb = (b + NBUF - 1) % NBUF
                pltpu.make_async_copy(y_hbm.at[src_v.at[i]], rows[b],
                                      gsem[b]).wait()
                pltpu.async_copy(rows[b], acc.at[dst_v.at[i]], ssem[b],
                                 add=True)
                pltpu.make_async_copy(rows[nb], acc.at[dst_v.at[i - 1]],
                                      ssem[nb]).wait()
                pltpu.async_copy(y_hbm.at[src_v.at[i + NBUF - 1]], rows[nb],
                                 gsem[nb])

        for i in range(NCHUNK - NBUF + 1, NCHUNK):
            b = i % NBUF
            pltpu.make_async_copy(y_hbm.at[src_v.at[i]], rows[b],
                                  gsem[b]).wait()
            pltpu.async_copy(rows[b], acc.at[dst_v.at[i]], ssem[b], add=True)

        for b in range(NBUF):
            i = NCHUNK - NBUF + b
            pltpu.make_async_copy(rows[b], acc.at[dst_v.at[i]],
                                  ssem[b]).wait()
        plsc.subcore_barrier()

        @pl.when(c == 0)
        def _():
            pltpu.sync_copy(acc.at[pl.ds(rbase, ROWS_PER_TILE)],
                            out0.at[pl.ds(rbase, ROWS_PER_TILE)])

            @pl.when(s == 0)
            def _():
                pltpu.sync_copy(acc.at[pl.ds(NS * ROWS_PER_TILE, REM_ROWS)],
                                out0.at[pl.ds(NS * ROWS_PER_TILE, REM_ROWS)])

        @pl.when(c == 1)
        def _():
            pltpu.sync_copy(acc.at[pl.ds(rbase, ROWS_PER_TILE)],
                            out1.at[pl.ds(rbase, ROWS_PER_TILE)])

            @pl.when(s == 0)
            def _():
                pltpu.sync_copy(acc.at[pl.ds(NS * ROWS_PER_TILE, REM_ROWS)],
                                out1.at[pl.ds(NS * ROWS_PER_TILE, REM_ROWS)])

    return segsum


_CB_ROWS = 312


@functools.lru_cache(maxsize=None)
def _make_sc_combine():
    mesh = plsc.VectorSubcoreMesh(core_axis_name="c", subcore_axis_name="s",
                                  num_cores=NC, num_subcores=NS)

    @functools.partial(
        pl.kernel,
        out_type=jax.ShapeDtypeStruct((N, D2), jnp.float32),
        mesh=mesh,
        compiler_params=pltpu.CompilerParams(use_tc_tiling_on_sc=False),
        scratch_types=[
            pltpu.VMEM((_CB_ROWS, D2), jnp.float32),
            pltpu.VMEM((_CB_ROWS, D2), jnp.float32),
            pltpu.VMEM((_CB_ROWS, D2), jnp.float32),
            pltpu.VMEM((D2,), jnp.float32),
            [pltpu.SemaphoreType.DMA for _ in range(3)],
        ],
    )
    def combine(y_hbm, q0_hbm, q1_hbm, b_hbm, out_hbm, yv, q0v, q1v, bv,
                sems):
        c = lax.axis_index("c")
        s = lax.axis_index("s")
        wid = s * NC + c

        pltpu.sync_copy(b_hbm, bv)
        segs = [(0, bv[pl.ds(0, 16)]), (16, bv[pl.ds(16, 16)]),
                (24, bv[pl.ds(24, 16)])]

        def do_block(r0, nrows):
            pltpu.async_copy(y_hbm.at[pl.ds(r0, nrows)], yv.at[pl.ds(0, nrows)], sems[0])
            pltpu.async_copy(q0_hbm.at[pl.ds(r0, nrows)], q0v.at[pl.ds(0, nrows)], sems[1])
            pltpu.async_copy(q1_hbm.at[pl.ds(r0, nrows)], q1v.at[pl.ds(0, nrows)], sems[2])
            pltpu.make_async_copy(y_hbm.at[pl.ds(r0, nrows)], yv.at[pl.ds(0, nrows)], sems[0]).wait()
            pltpu.make_async_copy(q0_hbm.at[pl.ds(r0, nrows)], q0v.at[pl.ds(0, nrows)], sems[1]).wait()
            pltpu.make_async_copy(q1_hbm.at[pl.ds(r0, nrows)], q1v.at[pl.ds(0, nrows)], sems[2]).wait()

            def row(i, carry):
                vals = [(off,
                         yv[i, pl.ds(off, 16)] + q0v[i, pl.ds(off, 16)]
                         + q1v[i, pl.ds(off, 16)] + bseg)
                        for off, bseg in segs]
                for off, v in vals:
                    yv[i, pl.ds(off, 16)] = v
                return carry

            lax.fori_loop(0, nrows, row, 0)
            pltpu.sync_copy(yv.at[pl.ds(0, nrows)],
                            out_hbm.at[pl.ds(r0, nrows)])

        do_block(wid * _CB_ROWS, _CB_ROWS)

        @pl.when(wid == 0)
        def _():
            do_block(NW * _CB_ROWS, N - NW * _CB_ROWS)

    return combine


_BLK = 2000


def _fuse_body(x_ref, p0_ref, p1_ref, w1_ref, b1_ref, w2_ref, o_ref):
    s = x_ref[...] + p0_ref[...] + p1_ref[...]
    z = jnp.dot(s, w1_ref[...], preferred_element_type=jnp.float32)
    z = jnp.maximum(z + b1_ref[...], 0.0)
    o_ref[...] = jnp.dot(z, w2_ref[...], preferred_element_type=jnp.float32)


def _tc_fuse_matmul(x, p0, p1, w1, b1, w2):
    m, k = x.shape
    n = w2.shape[1]
    return pl.pallas_call(
        _fuse_body,
        grid=(m // _BLK,),
        in_specs=[
            pl.BlockSpec((_BLK, k), lambda i: (i, 0)),
            pl.BlockSpec((_BLK, k), lambda i: (i, 0)),
            pl.BlockSpec((_BLK, k), lambda i: (i, 0)),
            pl.BlockSpec((k, k), lambda i: (0, 0)),
            pl.BlockSpec((1, k), lambda i: (0, 0)),
            pl.BlockSpec((k, n), lambda i: (0, 0)),
        ],
        out_specs=pl.BlockSpec((_BLK, n), lambda i: (i, 0)),
        out_shape=jax.ShapeDtypeStruct((m, n), jnp.float32),
    )(x, p0, p1, w1, b1, w2)


def kernel(features, edge_index, W1, b1, W2, b2):
    ei3 = edge_index.reshape(2, E // _CHUNK[D1], _CHUNK[D1])
    z1 = jnp.asarray(np.zeros((N, D1), np.float32))
    z2 = jnp.asarray(np.zeros((N, D2), np.float32))
    b1r = b1.reshape(1, D1)

    p0, p1 = _make_sc_segsum(D1)(features, ei3, z1)
    y2 = _tc_fuse_matmul(features, p0, p1, W1, b1r, W2)
    q0, q1 = _make_sc_segsum(D2)(y2, ei3, z2)
    return _make_sc_combine()(y2, q0, q1, b2)


# speedup vs baseline: 1.1952x; 1.0157x over previous
import functools

import jax
import jax.numpy as jnp
import numpy as np
from jax import lax
from jax.experimental import pallas as pl
from jax.experimental.pallas import tpu as pltpu
from jax.experimental.pallas import tpu_sc as plsc

N = 10000
E = 320000
D1 = 128
D2 = 40

NC = 2
NS = 16
NW = NC * NS
E_PER_W = E // NW
ROWS_PER_TILE = 624
REM_ROWS = N - NS * ROWS_PER_TILE

_CHUNK = {128: 40, 40: 40}
_NBUF = {128: 5, 40: 10}


@functools.lru_cache(maxsize=None)
def _make_sc_segsum(D):
    CHUNK = _CHUNK[D]
    NBUF = _NBUF[D]
    NCHUNK = E_PER_W // CHUNK
    mesh = plsc.VectorSubcoreMesh(core_axis_name="c", subcore_axis_name="s",
                                  num_cores=NC, num_subcores=NS)

    @functools.partial(
        pl.kernel,
        out_type=(
            jax.ShapeDtypeStruct((N, D), jnp.float32),
            jax.ShapeDtypeStruct((N, D), jnp.float32),
        ),
        mesh=mesh,
        compiler_params=pltpu.CompilerParams(use_tc_tiling_on_sc=False),
        scratch_types=[
            pltpu.VMEM((NCHUNK, CHUNK), jnp.int32),
            pltpu.VMEM((NCHUNK, CHUNK), jnp.int32),
            [pltpu.VMEM((CHUNK, D), jnp.float32) for _ in range(NBUF)],
            pltpu.VMEM_SHARED((N, D), jnp.float32),
            [pltpu.SemaphoreType.DMA for _ in range(NBUF)],
            [pltpu.SemaphoreType.DMA for _ in range(NBUF)],
            pltpu.SemaphoreType.DMA,
            pltpu.SemaphoreType.DMA,
        ],
    )
    def segsum(y_hbm, ei_hbm, zeros_hbm, out0, out1,
               src_v, dst_v, rows, acc, gsem, ssem, sem, zsem):
        c = lax.axis_index("c")
        s = lax.axis_index("s")
        wid = s * NC + c
        cbase = wid * NCHUNK

        pltpu.async_copy(ei_hbm.at[0, pl.ds(cbase, NCHUNK)], src_v, sem)
        pltpu.async_copy(ei_hbm.at[1, pl.ds(cbase, NCHUNK)], dst_v, sem)

        rbase = s * ROWS_PER_TILE
        pltpu.async_copy(zeros_hbm.at[pl.ds(rbase, ROWS_PER_TILE)],
                         acc.at[pl.ds(rbase, ROWS_PER_TILE)], zsem)

        @pl.when(s == 0)
        def _():
            pltpu.async_copy(zeros_hbm.at[pl.ds(NS * ROWS_PER_TILE, REM_ROWS)],
                             acc.at[pl.ds(NS * ROWS_PER_TILE, REM_ROWS)], zsem)

        pltpu.make_async_copy(ei_hbm.at[0, pl.ds(cbase, NCHUNK)], src_v,
                              sem).wait()
        pltpu.make_async_copy(ei_hbm.at[1, pl.ds(cbase, NCHUNK)], dst_v,
                              sem).wait()

        for b in range(NBUF - 1):
            pltpu.async_copy(y_hbm.at[src_v.at[b]], rows[b], gsem[b])

        pltpu.make_async_copy(zeros_hbm.at[pl.ds(rbase, ROWS_PER_TILE)],
                              acc.at[pl.ds(rbase, ROWS_PER_TILE)], zsem).wait()

        @pl.when(s == 0)
        def _():
            pltpu.make_async_copy(
                zeros_hbm.at[pl.ds(NS * ROWS_PER_TILE, REM_ROWS)],
                acc.at[pl.ds(NS * ROWS_PER_TILE, REM_ROWS)], zsem).wait()

        plsc.subcore_barrier()

        pltpu.make_async_copy(y_hbm.at[src_v.at[0]], rows[0], gsem[0]).wait()
        pltpu.async_copy(rows[0], acc.at[dst_v.at[0]], ssem[0], add=True)
        pltpu.async_copy(y_hbm.at[src_v.at[NBUF - 1]], rows[NBUF - 1],
                         gsem[NBUF - 1])

        @pl.loop(1, NCHUNK - NBUF + 1, step=NBUF)
        def _(j):
            for t in range(NBUF):
                i = j + t
                b = (1 + t) % NBUF
                nb = (b + NBUF - 1) % NBUF
                pltpu.make_async_copy(y_hbm.at[src_v.at[i]], rows[b],
                                      gsem[b]).wait()
                pltpu.async_copy(rows[b], acc.at[dst_v.at[i]], ssem[b],
                                 add=True)
                pltpu.make_async_copy(rows[nb], acc.at[dst_v.at[i - 1]],
                                      ssem[nb]).wait()
                pltpu.async_copy(y_hbm.at[src_v.at[i + NBUF - 1]], rows[nb],
                                 gsem[nb])

        for i in range(NCHUNK - NBUF + 1, NCHUNK):
            b = i % NBUF
            pltpu.make_async_copy(y_hbm.at[src_v.at[i]], rows[b],
                                  gsem[b]).wait()
            pltpu.async_copy(rows[b], acc.at[dst_v.at[i]], ssem[b], add=True)

        for b in range(NBUF):
            i = NCHUNK - NBUF + b
            pltpu.make_async_copy(rows[b], acc.at[dst_v.at[i]],
                                  ssem[b]).wait()
        plsc.subcore_barrier()

        @pl.when(c == 0)
        def _():
            pltpu.sync_copy(acc.at[pl.ds(rbase, ROWS_PER_TILE)],
                            out0.at[pl.ds(rbase, ROWS_PER_TILE)])

            @pl.when(s == 0)
            def _():
                pltpu.sync_copy(acc.at[pl.ds(NS * ROWS_PER_TILE, REM_ROWS)],
                                out0.at[pl.ds(NS * ROWS_PER_TILE, REM_ROWS)])

        @pl.when(c == 1)
        def _():
            pltpu.sync_copy(acc.at[pl.ds(rbase, ROWS_PER_TILE)],
                            out1.at[pl.ds(rbase, ROWS_PER_TILE)])

            @pl.when(s == 0)
            def _():
                pltpu.sync_copy(acc.at[pl.ds(NS * ROWS_PER_TILE, REM_ROWS)],
                                out1.at[pl.ds(NS * ROWS_PER_TILE, REM_ROWS)])

    return segsum


_CB_ROWS = 312


@functools.lru_cache(maxsize=None)
def _make_sc_combine():
    mesh = plsc.VectorSubcoreMesh(core_axis_name="c", subcore_axis_name="s",
                                  num_cores=NC, num_subcores=NS)

    @functools.partial(
        pl.kernel,
        out_type=jax.ShapeDtypeStruct((N, D2), jnp.float32),
        mesh=mesh,
        compiler_params=pltpu.CompilerParams(use_tc_tiling_on_sc=False),
        scratch_types=[
            pltpu.VMEM((_CB_ROWS, D2), jnp.float32),
            pltpu.VMEM((_CB_ROWS, D2), jnp.float32),
            pltpu.VMEM((_CB_ROWS, D2), jnp.float32),
            pltpu.VMEM((D2,), jnp.float32),
            [pltpu.SemaphoreType.DMA for _ in range(3)],
        ],
    )
    def combine(y_hbm, q0_hbm, q1_hbm, b_hbm, out_hbm, yv, q0v, q1v, bv,
                sems):
        c = lax.axis_index("c")
        s = lax.axis_index("s")
        wid = s * NC + c

        pltpu.sync_copy(b_hbm, bv)
        segs = [(0, bv[pl.ds(0, 16)]), (16, bv[pl.ds(16, 16)]),
                (24, bv[pl.ds(24, 16)])]

        def do_block(r0, nrows):
            pltpu.async_copy(y_hbm.at[pl.ds(r0, nrows)], yv.at[pl.ds(0, nrows)], sems[0])
            pltpu.async_copy(q0_hbm.at[pl.ds(r0, nrows)], q0v.at[pl.ds(0, nrows)], sems[1])
            pltpu.async_copy(q1_hbm.at[pl.ds(r0, nrows)], q1v.at[pl.ds(0, nrows)], sems[2])
            pltpu.make_async_copy(y_hbm.at[pl.ds(r0, nrows)], yv.at[pl.ds(0, nrows)], sems[0]).wait()
            pltpu.make_async_copy(q0_hbm.at[pl.ds(r0, nrows)], q0v.at[pl.ds(0, nrows)], sems[1]).wait()
            pltpu.make_async_copy(q1_hbm.at[pl.ds(r0, nrows)], q1v.at[pl.ds(0, nrows)], sems[2]).wait()

            def row(i, carry):
                vals = [(off,
                         yv[i, pl.ds(off, 16)] + q0v[i, pl.ds(off, 16)]
                         + q1v[i, pl.ds(off, 16)] + bseg)
                        for off, bseg in segs]
                for off, v in vals:
                    yv[i, pl.ds(off, 16)] = v
                return carry

            lax.fori_loop(0, nrows, row, 0)
            pltpu.sync_copy(yv.at[pl.ds(0, nrows)],
                            out_hbm.at[pl.ds(r0, nrows)])

        do_block(wid * _CB_ROWS, _CB_ROWS)

        @pl.when(wid == 0)
        def _():
            do_block(NW * _CB_ROWS, N - NW * _CB_ROWS)

    return combine


_BLK = 2000


def _fuse_body(x_ref, p0_ref, p1_ref, w1_ref, b1_ref, w2_ref, o_ref):
    s = x_ref[...] + p0_ref[...] + p1_ref[...]
    z = jnp.dot(s, w1_ref[...], preferred_element_type=jnp.float32)
    z = jnp.maximum(z + b1_ref[...], 0.0)
    o_ref[...] = jnp.dot(z, w2_ref[...], preferred_element_type=jnp.float32)


def _tc_fuse_matmul(x, p0, p1, w1, b1, w2):
    m, k = x.shape
    n = w2.shape[1]
    return pl.pallas_call(
        _fuse_body,
        grid=(m // _BLK,),
        in_specs=[
            pl.BlockSpec((_BLK, k), lambda i: (i, 0)),
            pl.BlockSpec((_BLK, k), lambda i: (i, 0)),
            pl.BlockSpec((_BLK, k), lambda i: (i, 0)),
            pl.BlockSpec((k, k), lambda i: (0, 0)),
            pl.BlockSpec((1, k), lambda i: (0, 0)),
            pl.BlockSpec((k, n), lambda i: (0, 0)),
        ],
        out_specs=pl.BlockSpec((_BLK, n), lambda i: (i, 0)),
        out_shape=jax.ShapeDtypeStruct((m, n), jnp.float32),
    )(x, p0, p1, w1, b1, w2)


def kernel(features, edge_index, W1, b1, W2, b2):
    ei3 = edge_index.reshape(2, E // _CHUNK[D1], _CHUNK[D1])
    z1 = jnp.asarray(np.zeros((N, D1), np.float32))
    z2 = jnp.asarray(np.zeros((N, D2), np.float32))
    b1r = b1.reshape(1, D1)

    p0, p1 = _make_sc_segsum(D1)(features, ei3, z1)
    y2 = _tc_fuse_matmul(features, p0, p1, W1, b1r, W2)
    q0, q1 = _make_sc_segsum(D2)(y2, ei3, z2)
    return _make_sc_combine()(y2, q0, q1, b2)
